# Initial kernel scaffold; baseline (speedup 1.0000x reference)
import jax
import jax.numpy as jnp
from jax.experimental import pallas as pl


def kernel(x, edge_index, edge_attr, node_time, node_batch, seed_time, n_id, emb_table, time_w, W0, We0, a_src0, a_dst0, a_e0, W1, We1, a_src1, a_dst1, a_e1, head_w, head_b):
    raise NotImplementedError("write your pallas kernel here")


# speedup vs baseline: 6.9881x; 6.9881x over previous
import functools
import math

import jax
import jax.numpy as jnp
import numpy as np
from jax import lax
from jax.experimental import pallas as pl
from jax.experimental.pallas import tpu as pltpu
from jax.experimental.pallas import tpu_sc as plsc

N, E, C, CE, B, VOC = 10000, 320000, 128, 16, 512, 100000
NP = 10240
NC, NS, L = 2, 16, 16
NW = NC * NS
RT = NP // NW
ET = E // NW
CK = 80
CH = ET // CK
AC = 144
RS = NP // NS

_mesh = plsc.VectorSubcoreMesh(core_axis_name="c", subcore_axis_name="s")
_f32 = jnp.float32
_sc_params = pltpu.CompilerParams(
    use_tc_tiling_on_sc=False, needs_layout_passes=False)


def _prelude_body(nid3, nt3, nb3, st_hbm, emb_hbm, emb_out, rel8_out,
                  idx_v, rows_v, nt_v, nb_v, st_v, rtmp, rel8_v, sem):
    wid = lax.axis_index("s") * NC + lax.axis_index("c")
    base = wid * RT
    pltpu.sync_copy(nid3.at[wid], idx_v)
    pltpu.sync_copy(nt3.at[wid], nt_v)
    pltpu.sync_copy(nb3.at[wid], nb_v)
    pltpu.sync_copy(st_hbm, st_v)
    for j in range(RT // CK):
        pltpu.async_copy(emb_hbm.at[idx_v.at[j]],
                         rows_v.at[pl.ds(j * CK, CK)], sem).wait()
    pltpu.sync_copy(rows_v, emb_out.at[pl.ds(base, RT)])
    inv = _f32(1.0 / 86400.0)
    lane = lax.iota(jnp.int32, L)
    for t in range(RT // L):
        nb = nb_v[pl.ds(t * L, L)]
        s = plsc.load_gather(st_v, [nb])
        r = (s - nt_v[pl.ds(t * L, L)]) * inv
        rtmp[...] = r
        for q in range(8):
            rq = plsc.load_gather(rtmp, [2 * q + lane // 8])
            rel8_v[pl.ds(t * 128 + q * L, L)] = rq
    pltpu.sync_copy(rel8_v, rel8_out.at[pl.ds(base * 8, RT * 8)])


_prelude = functools.partial(
    pl.kernel, _prelude_body, mesh=_mesh, compiler_params=_sc_params,
    out_type=(jax.ShapeDtypeStruct((NP, C), _f32),
              jax.ShapeDtypeStruct((NP * 8,), _f32)),
    scratch_types=[
        pltpu.VMEM((RT // CK, CK), jnp.int32),
        pltpu.VMEM((RT, C), _f32),
        pltpu.VMEM((RT,), _f32),
        pltpu.VMEM((RT,), jnp.int32),
        pltpu.VMEM((B,), _f32),
        pltpu.VMEM((L,), _f32),
        pltpu.VMEM((RT * 8,), _f32),
        pltpu.SemaphoreType.DMA,
    ],
)()


CK1 = CK + 1
JUNK = NP - 1


def _edge_body(hs_hbm, ss_hbm, sd_hbm, gidx3, src3, dst3, dful3, eatt3,
               agg_out, sst, sdt, gib, srcb, dstb, dfb, eb, wbuf, scb,
               agg_sh, sem):
    cid = lax.axis_index("c")
    sid = lax.axis_index("s")
    wid = sid * NC + cid
    zero = jnp.zeros((L,), _f32)
    for r in range(CK1):
        for k in range(AC // L):
            scb[r, pl.ds(k * L, L)] = zero
    for b in range(RS // CK):
        pltpu.sync_copy(scb.at[pl.ds(0, CK)],
                        agg_sh.at[pl.ds(sid * RS + b * CK, CK)])
    plsc.subcore_barrier()
    pltpu.sync_copy(ss_hbm, sst)
    pltpu.sync_copy(sd_hbm, sdt)

    def chunk(j, carry):
        pltpu.sync_copy(gidx3.at[wid, j], gib)
        pltpu.sync_copy(src3.at[wid, j], srcb)
        pltpu.sync_copy(dst3.at[wid, j], dstb)
        pltpu.sync_copy(dful3.at[wid, j], dfb)
        pltpu.sync_copy(eatt3.at[wid, j], eb)
        pltpu.async_copy(hs_hbm.at[gib], scb, sem).wait()
        for g in range(CK // L):
            sv = srcb[pl.ds(g * L, L)]
            dv = dstb[pl.ds(g * L, L)]
            ev = eb[pl.ds(g * L, L)]
            lg = plsc.load_gather(sst, [sv]) + plsc.load_gather(sdt, [dv]) + ev
            lg = jnp.maximum(lg, lg * _f32(0.2))
            wbuf[pl.ds(g * L, L)] = jnp.exp(lg)

        def scale(r, c):
            wsp = plsc.load_gather(wbuf, [jnp.full((L,), r, jnp.int32)])
            for k in range(AC // L):
                scb[r + 1, pl.ds(k * L, L)] = scb[r + 1, pl.ds(k * L, L)] * wsp
            return c

        lax.fori_loop(0, CK, scale, 0)
        pltpu.sync_copy(scb, agg_sh.at[dfb], add=True)
        return carry

    lax.fori_loop(0, CH, chunk, 0)
    plsc.subcore_barrier()
    for b in range(RS // CK):
        s0 = sid * RS + b * CK
        pltpu.sync_copy(agg_sh.at[pl.ds(s0, CK)], scb.at[pl.ds(0, CK)])
        pltpu.sync_copy(scb.at[pl.ds(0, CK)], agg_out.at[cid, pl.ds(s0, CK)])


_edge = functools.partial(
    pl.kernel, _edge_body, mesh=_mesh, compiler_params=_sc_params,
    out_type=jax.ShapeDtypeStruct((NC, NP, AC), _f32),
    scratch_types=[
        pltpu.VMEM((NP,), _f32),
        pltpu.VMEM((NP,), _f32),
        pltpu.VMEM((CK1,), jnp.int32),
        pltpu.VMEM((CK,), jnp.int32),
        pltpu.VMEM((CK,), jnp.int32),
        pltpu.VMEM((CK1,), jnp.int32),
        pltpu.VMEM((CK,), _f32),
        pltpu.VMEM((CK,), _f32),
        pltpu.VMEM((CK1, AC), _f32),
        pltpu.VMEM_SHARED((NP, AC), _f32),
        pltpu.SemaphoreType.DMA,
    ],
)()


RB = 512

def _t0_body(x_r, emb_r, rel8_r, tw_r, w0_r, asrc_r, adst_r,
             hs_r, ss_r, sd_r):
    col = lax.broadcasted_iota(jnp.int32, (1, 8), 1)
    f8 = jnp.exp2((col % 4).astype(_f32))
    ph = jnp.where(col >= 4, _f32(0.5 * math.pi), _f32(0.0))
    tf = jnp.sin(rel8_r[...] * f8 + ph)
    h = x_r[...] + emb_r[...] + jnp.dot(tf, tw_r[...],
                                        preferred_element_type=_f32)
    hs = jnp.dot(h, w0_r[...], preferred_element_type=_f32)
    ac_col = lax.broadcasted_iota(jnp.int32, (1, AC), 1)
    hs_r[...] = (jnp.pad(hs, ((0, 0), (0, AC - C)))
                 + jnp.where(ac_col == C, _f32(1.0), _f32(0.0)))
    ss_r[...] = jnp.sum(hs * asrc_r[...], axis=1, keepdims=True)
    sd_r[...] = jnp.sum(hs * adst_r[...], axis=1, keepdims=True)


def _t0(xp, emb, rel8, tw, w0, asrc, adst):
    g = NP // RB
    return pl.pallas_call(
        _t0_body,
        grid=(g,),
        in_specs=[
            pl.BlockSpec((RB, C), lambda i: (i, 0)),
            pl.BlockSpec((RB, C), lambda i: (i, 0)),
            pl.BlockSpec((RB, 8), lambda i: (i, 0)),
            pl.BlockSpec((8, C), lambda i: (0, 0)),
            pl.BlockSpec((C, C), lambda i: (0, 0)),
            pl.BlockSpec((1, C), lambda i: (0, 0)),
            pl.BlockSpec((1, C), lambda i: (0, 0)),
        ],
        out_specs=[
            pl.BlockSpec((RB, AC), lambda i: (i, 0)),
            pl.BlockSpec((RB, 1), lambda i: (i, 0)),
            pl.BlockSpec((RB, 1), lambda i: (i, 0)),
        ],
        out_shape=[
            jax.ShapeDtypeStruct((NP, AC), _f32),
            jax.ShapeDtypeStruct((NP, 1), _f32),
            jax.ShapeDtypeStruct((NP, 1), _f32),
        ],
    )(xp, emb, rel8, tw, w0, asrc, adst)


def _t1_body(a0_r, a1_r, hs0_r, w1_r, asrc_r, adst_r, hs_r, ss_r, sd_r):
    num = a0_r[:, :C] + a1_r[:, :C]
    den = a0_r[:, C:C + 1] + a1_r[:, C:C + 1]
    h = jnp.maximum(num / (den + _f32(1e-16)) + hs0_r[:, :C], _f32(0.0))
    hs = jnp.dot(h, w1_r[...], preferred_element_type=_f32)
    ac_col = lax.broadcasted_iota(jnp.int32, (1, AC), 1)
    hs_r[...] = (jnp.pad(hs, ((0, 0), (0, AC - C)))
                 + jnp.where(ac_col == C, _f32(1.0), _f32(0.0)))
    ss_r[...] = jnp.sum(hs * asrc_r[...], axis=1, keepdims=True)
    sd_r[...] = jnp.sum(hs * adst_r[...], axis=1, keepdims=True)


def _t1(a0, a1, hs0, w1, asrc, adst):
    g = NP // RB
    return pl.pallas_call(
        _t1_body,
        grid=(g,),
        in_specs=[
            pl.BlockSpec((RB, AC), lambda i: (i, 0)),
            pl.BlockSpec((RB, AC), lambda i: (i, 0)),
            pl.BlockSpec((RB, AC), lambda i: (i, 0)),
            pl.BlockSpec((C, C), lambda i: (0, 0)),
            pl.BlockSpec((1, C), lambda i: (0, 0)),
            pl.BlockSpec((1, C), lambda i: (0, 0)),
        ],
        out_specs=[
            pl.BlockSpec((RB, AC), lambda i: (i, 0)),
            pl.BlockSpec((RB, 1), lambda i: (i, 0)),
            pl.BlockSpec((RB, 1), lambda i: (i, 0)),
        ],
        out_shape=[
            jax.ShapeDtypeStruct((NP, AC), _f32),
            jax.ShapeDtypeStruct((NP, 1), _f32),
            jax.ShapeDtypeStruct((NP, 1), _f32),
        ],
    )(a0, a1, hs0, w1, asrc, adst)


def _t2_body(a0_r, a1_r, hs1_r, hw_r, hb_r, out_r):
    num = a0_r[:, :C] + a1_r[:, :C]
    den = a0_r[:, C:C + 1] + a1_r[:, C:C + 1]
    h = jnp.maximum(num / (den + _f32(1e-16)) + hs1_r[:, :C], _f32(0.0))
    z = jnp.dot(h, hw_r[...], preferred_element_type=_f32) + hb_r[...]
    out_r[...] = jax.nn.sigmoid(z)


def _t2(a0, a1, hs1b, hw, hb):
    return pl.pallas_call(
        _t2_body,
        out_shape=jax.ShapeDtypeStruct((B, 1), _f32),
    )(a0, a1, hs1b, hw, hb)


E8 = E // 8


def _te_body(x8_r, m_r, o_r):
    o_r[...] = jnp.dot(x8_r[...], m_r[...], preferred_element_type=_f32)


def _te(x8, m):
    g = 10
    rb = E8 // g
    return pl.pallas_call(
        _te_body,
        grid=(g,),
        in_specs=[
            pl.BlockSpec((rb, C), lambda i: (i, 0)),
            pl.BlockSpec((C, CE), lambda i: (0, 0)),
        ],
        out_specs=pl.BlockSpec((rb, CE), lambda i: (i, 0)),
        out_shape=jax.ShapeDtypeStruct((E8, CE), _f32),
    )(x8, m)


def kernel(x, edge_index, edge_attr, node_time, node_batch, seed_time, n_id,
           emb_table, time_w, W0, We0, a_src0, a_dst0, a_e0, W1, We1, a_src1,
           a_dst1, a_e1, head_w, head_b):
    pad_n = NP - N
    xp = jnp.pad(x, ((0, pad_n), (0, 0)))
    ntp = jnp.pad(node_time, (0, pad_n))
    nbp = jnp.pad(node_batch, (0, pad_n)).astype(jnp.int32)
    nidp = jnp.pad(n_id, (0, pad_n)).astype(jnp.int32)

    nid3 = nidp.reshape(NW, RT // CK, CK)
    nt3 = ntp.reshape(NW, RT)
    nb3 = nbp.reshape(NW, RT)

    emb, rel8f = _prelude(nid3, nt3, nb3, seed_time, emb_table)
    rel8 = rel8f.reshape(NP, 8)

    wa = jnp.stack([We0 @ a_e0, We1 @ a_e1], axis=1)
    m = jnp.kron(jnp.eye(8, dtype=_f32), wa)
    x8 = edge_attr.reshape(E8, C)
    e2 = _te(x8, m).reshape(E8, 8, 2).reshape(E, 2)

    src = edge_index[0].astype(jnp.int32)
    dst = edge_index[1].astype(jnp.int32)
    src3 = src.reshape(NW, CH, CK)
    dst3 = dst.reshape(NW, CH, CK)
    gidx3 = jnp.concatenate([src3[:, :, :1], src3], axis=2)
    dful3 = jnp.concatenate([jnp.full_like(dst3[:, :, :1], JUNK), dst3],
                            axis=2)
    ea0 = e2[:, 0].reshape(NW, CH, CK)
    ea1 = e2[:, 1].reshape(NW, CH, CK)

    asrc0 = a_src0.reshape(1, C)
    adst0 = a_dst0.reshape(1, C)
    asrc1 = a_src1.reshape(1, C)
    adst1 = a_dst1.reshape(1, C)

    hs0, ss0, sd0 = _t0(xp, emb, rel8, time_w, W0, asrc0, adst0)
    agg0 = _edge(hs0, ss0.reshape(NP), sd0.reshape(NP), gidx3, src3, dst3,
                 dful3, ea0)
    hs1, ss1, sd1 = _t1(agg0[0], agg0[1], hs0, W1, asrc1, adst1)
    agg1 = _edge(hs1, ss1.reshape(NP), sd1.reshape(NP), gidx3, src3, dst3,
                 dful3, ea1)
    return _t2(agg1[0, :B], agg1[1, :B], hs1[:B], head_w,
               head_b.reshape(1, 1))


# speedup vs baseline: 8.2310x; 1.1779x over previous
import functools
import math

import jax
import jax.numpy as jnp
import numpy as np
from jax import lax
from jax.experimental import pallas as pl
from jax.experimental.pallas import tpu as pltpu
from jax.experimental.pallas import tpu_sc as plsc

N, E, C, CE, B, VOC = 10000, 320000, 128, 16, 512, 100000
NP = 10240
NC, NS, L = 2, 16, 16
NW = NC * NS
RT = NP // NW
ET = E // NW
CK = 80
CH = ET // CK
AC = 144
RS = NP // NS

_mesh = plsc.VectorSubcoreMesh(core_axis_name="c", subcore_axis_name="s")
_f32 = jnp.float32
_sc_params = pltpu.CompilerParams(
    use_tc_tiling_on_sc=False, needs_layout_passes=False)


def _prelude_body(nid3, nt3, nb3, st_hbm, emb_hbm, emb_out, rel8_out,
                  idx_v, rows_v, nt_v, nb_v, st_v, rtmp, rel8_v, sem):
    wid = lax.axis_index("s") * NC + lax.axis_index("c")
    base = wid * RT
    pltpu.sync_copy(nid3.at[wid], idx_v)
    pltpu.sync_copy(nt3.at[wid], nt_v)
    pltpu.sync_copy(nb3.at[wid], nb_v)
    pltpu.sync_copy(st_hbm, st_v)
    for j in range(RT // CK):
        pltpu.async_copy(emb_hbm.at[idx_v.at[j]],
                         rows_v.at[pl.ds(j * CK, CK)], sem).wait()
    pltpu.sync_copy(rows_v, emb_out.at[pl.ds(base, RT)])
    inv = _f32(1.0 / 86400.0)
    lane = lax.iota(jnp.int32, L)
    for t in range(RT // L):
        nb = nb_v[pl.ds(t * L, L)]
        s = plsc.load_gather(st_v, [nb])
        r = (s - nt_v[pl.ds(t * L, L)]) * inv
        rtmp[...] = r
        for q in range(8):
            rq = plsc.load_gather(rtmp, [2 * q + lane // 8])
            rel8_v[pl.ds(t * 128 + q * L, L)] = rq
    pltpu.sync_copy(rel8_v, rel8_out.at[pl.ds(base * 8, RT * 8)])


_prelude = functools.partial(
    pl.kernel, _prelude_body, mesh=_mesh, compiler_params=_sc_params,
    out_type=(jax.ShapeDtypeStruct((NP, C), _f32),
              jax.ShapeDtypeStruct((NP * 8,), _f32)),
    scratch_types=[
        pltpu.VMEM((RT // CK, CK), jnp.int32),
        pltpu.VMEM((RT, C), _f32),
        pltpu.VMEM((RT,), _f32),
        pltpu.VMEM((RT,), jnp.int32),
        pltpu.VMEM((B,), _f32),
        pltpu.VMEM((L,), _f32),
        pltpu.VMEM((RT * 8,), _f32),
        pltpu.SemaphoreType.DMA,
    ],
)()


CK1 = CK + 1
JUNK = NP - 1


def _edge_body(hs_hbm, ss_hbm, sd_hbm, gidx3, src3, dst3, dful3, eatt3,
               agg_out, sst, sdt, gib, srcb, dstb, dfb, eb, wbuf, scb,
               agg_sh, sem):
    cid = lax.axis_index("c")
    sid = lax.axis_index("s")
    wid = sid * NC + cid
    zero = jnp.zeros((L,), _f32)
    for r in range(CK1):
        for k in range(AC // L):
            scb[r, pl.ds(k * L, L)] = zero
    for b in range(RS // CK):
        pltpu.sync_copy(scb.at[pl.ds(0, CK)],
                        agg_sh.at[pl.ds(sid * RS + b * CK, CK)])
    plsc.subcore_barrier()
    pltpu.sync_copy(ss_hbm, sst)
    pltpu.sync_copy(sd_hbm, sdt)

    def chunk(j, carry):
        pltpu.sync_copy(gidx3.at[wid, j], gib)
        cp = pltpu.async_copy(hs_hbm.at[gib], scb, sem)
        pltpu.sync_copy(src3.at[wid, j], srcb)
        pltpu.sync_copy(dst3.at[wid, j], dstb)
        pltpu.sync_copy(dful3.at[wid, j], dfb)
        pltpu.sync_copy(eatt3.at[wid, j], eb)
        for g in range(CK // L):
            sv = srcb[pl.ds(g * L, L)]
            dv = dstb[pl.ds(g * L, L)]
            ev = eb[pl.ds(g * L, L)]
            lg = plsc.load_gather(sst, [sv]) + plsc.load_gather(sdt, [dv]) + ev
            lg = jnp.maximum(lg, lg * _f32(0.2))
            wbuf[pl.ds(g * L, L)] = jnp.exp(lg)
        cp.wait()

        def scale(r, c):
            wsp = plsc.load_gather(wbuf, [jnp.full((L,), r, jnp.int32)])
            for k in range(AC // L):
                scb[r + 1, pl.ds(k * L, L)] = scb[r + 1, pl.ds(k * L, L)] * wsp
            return c

        lax.fori_loop(0, CK, scale, 0)
        pltpu.sync_copy(scb, agg_sh.at[dfb], add=True)
        return carry

    lax.fori_loop(0, CH, chunk, 0)
    plsc.subcore_barrier()
    for b in range(RS // CK):
        s0 = sid * RS + b * CK
        pltpu.sync_copy(agg_sh.at[pl.ds(s0, CK)], scb.at[pl.ds(0, CK)])
        pltpu.sync_copy(scb.at[pl.ds(0, CK)], agg_out.at[cid, pl.ds(s0, CK)])


_edge = functools.partial(
    pl.kernel, _edge_body, mesh=_mesh, compiler_params=_sc_params,
    out_type=jax.ShapeDtypeStruct((NC, NP, AC), _f32),
    scratch_types=[
        pltpu.VMEM((NP,), _f32),
        pltpu.VMEM((NP,), _f32),
        pltpu.VMEM((CK1,), jnp.int32),
        pltpu.VMEM((CK,), jnp.int32),
        pltpu.VMEM((CK,), jnp.int32),
        pltpu.VMEM((CK1,), jnp.int32),
        pltpu.VMEM((CK,), _f32),
        pltpu.VMEM((CK,), _f32),
        pltpu.VMEM((CK1, AC), _f32),
        pltpu.VMEM_SHARED((NP, AC), _f32),
        pltpu.SemaphoreType.DMA,
    ],
)()


RB = 512

def _t0_body(x_r, emb_r, rel8_r, tw_r, w0_r, asrc_r, adst_r,
             hs_r, ss_r, sd_r):
    col = lax.broadcasted_iota(jnp.int32, (1, 8), 1)
    f8 = jnp.exp2((col % 4).astype(_f32))
    ph = jnp.where(col >= 4, _f32(0.5 * math.pi), _f32(0.0))
    tf = jnp.sin(rel8_r[...] * f8 + ph)
    h = x_r[...] + emb_r[...] + jnp.dot(tf, tw_r[...],
                                        preferred_element_type=_f32)
    hs = jnp.dot(h, w0_r[...], preferred_element_type=_f32)
    ac_col = lax.broadcasted_iota(jnp.int32, (1, AC), 1)
    hs_r[...] = (jnp.pad(hs, ((0, 0), (0, AC - C)))
                 + jnp.where(ac_col == C, _f32(1.0), _f32(0.0)))
    ss_r[...] = jnp.sum(hs * asrc_r[...], axis=1, keepdims=True)
    sd_r[...] = jnp.sum(hs * adst_r[...], axis=1, keepdims=True)


def _t0(xp, emb, rel8, tw, w0, asrc, adst):
    g = NP // RB
    return pl.pallas_call(
        _t0_body,
        grid=(g,),
        in_specs=[
            pl.BlockSpec((RB, C), lambda i: (i, 0)),
            pl.BlockSpec((RB, C), lambda i: (i, 0)),
            pl.BlockSpec((RB, 8), lambda i: (i, 0)),
            pl.BlockSpec((8, C), lambda i: (0, 0)),
            pl.BlockSpec((C, C), lambda i: (0, 0)),
            pl.BlockSpec((1, C), lambda i: (0, 0)),
            pl.BlockSpec((1, C), lambda i: (0, 0)),
        ],
        out_specs=[
            pl.BlockSpec((RB, AC), lambda i: (i, 0)),
            pl.BlockSpec((RB, 1), lambda i: (i, 0)),
            pl.BlockSpec((RB, 1), lambda i: (i, 0)),
        ],
        out_shape=[
            jax.ShapeDtypeStruct((NP, AC), _f32),
            jax.ShapeDtypeStruct((NP, 1), _f32),
            jax.ShapeDtypeStruct((NP, 1), _f32),
        ],
    )(xp, emb, rel8, tw, w0, asrc, adst)


def _t1_body(a0_r, a1_r, hs0_r, w1_r, asrc_r, adst_r, hs_r, ss_r, sd_r):
    num = a0_r[:, :C] + a1_r[:, :C]
    den = a0_r[:, C:C + 1] + a1_r[:, C:C + 1]
    h = jnp.maximum(num / (den + _f32(1e-16)) + hs0_r[:, :C], _f32(0.0))
    hs = jnp.dot(h, w1_r[...], preferred_element_type=_f32)
    ac_col = lax.broadcasted_iota(jnp.int32, (1, AC), 1)
    hs_r[...] = (jnp.pad(hs, ((0, 0), (0, AC - C)))
                 + jnp.where(ac_col == C, _f32(1.0), _f32(0.0)))
    ss_r[...] = jnp.sum(hs * asrc_r[...], axis=1, keepdims=True)
    sd_r[...] = jnp.sum(hs * adst_r[...], axis=1, keepdims=True)


def _t1(a0, a1, hs0, w1, asrc, adst):
    g = NP // RB
    return pl.pallas_call(
        _t1_body,
        grid=(g,),
        in_specs=[
            pl.BlockSpec((RB, AC), lambda i: (i, 0)),
            pl.BlockSpec((RB, AC), lambda i: (i, 0)),
            pl.BlockSpec((RB, AC), lambda i: (i, 0)),
            pl.BlockSpec((C, C), lambda i: (0, 0)),
            pl.BlockSpec((1, C), lambda i: (0, 0)),
            pl.BlockSpec((1, C), lambda i: (0, 0)),
        ],
        out_specs=[
            pl.BlockSpec((RB, AC), lambda i: (i, 0)),
            pl.BlockSpec((RB, 1), lambda i: (i, 0)),
            pl.BlockSpec((RB, 1), lambda i: (i, 0)),
        ],
        out_shape=[
            jax.ShapeDtypeStruct((NP, AC), _f32),
            jax.ShapeDtypeStruct((NP, 1), _f32),
            jax.ShapeDtypeStruct((NP, 1), _f32),
        ],
    )(a0, a1, hs0, w1, asrc, adst)


def _t2_body(a0_r, a1_r, hs1_r, hw_r, hb_r, out_r):
    num = a0_r[:, :C] + a1_r[:, :C]
    den = a0_r[:, C:C + 1] + a1_r[:, C:C + 1]
    h = jnp.maximum(num / (den + _f32(1e-16)) + hs1_r[:, :C], _f32(0.0))
    z = jnp.dot(h, hw_r[...], preferred_element_type=_f32) + hb_r[...]
    out_r[...] = jax.nn.sigmoid(z)


def _t2(a0, a1, hs1b, hw, hb):
    return pl.pallas_call(
        _t2_body,
        out_shape=jax.ShapeDtypeStruct((B, 1), _f32),
    )(a0, a1, hs1b, hw, hb)


E8 = E // 8


def _te_body(x8_r, m_r, o_r):
    o_r[...] = jnp.dot(x8_r[...], m_r[...], preferred_element_type=_f32)


def _te(x8, m):
    g = 10
    rb = E8 // g
    return pl.pallas_call(
        _te_body,
        grid=(g,),
        in_specs=[
            pl.BlockSpec((rb, C), lambda i: (i, 0)),
            pl.BlockSpec((C, CE), lambda i: (0, 0)),
        ],
        out_specs=pl.BlockSpec((rb, CE), lambda i: (i, 0)),
        out_shape=jax.ShapeDtypeStruct((E8, CE), _f32),
    )(x8, m)


def kernel(x, edge_index, edge_attr, node_time, node_batch, seed_time, n_id,
           emb_table, time_w, W0, We0, a_src0, a_dst0, a_e0, W1, We1, a_src1,
           a_dst1, a_e1, head_w, head_b):
    pad_n = NP - N
    xp = jnp.pad(x, ((0, pad_n), (0, 0)))
    ntp = jnp.pad(node_time, (0, pad_n))
    nbp = jnp.pad(node_batch, (0, pad_n)).astype(jnp.int32)
    nidp = jnp.pad(n_id, (0, pad_n)).astype(jnp.int32)

    nid3 = nidp.reshape(NW, RT // CK, CK)
    nt3 = ntp.reshape(NW, RT)
    nb3 = nbp.reshape(NW, RT)

    emb, rel8f = _prelude(nid3, nt3, nb3, seed_time, emb_table)
    rel8 = rel8f.reshape(NP, 8)

    wa = jnp.stack([We0 @ a_e0, We1 @ a_e1], axis=1)
    m = jnp.kron(jnp.eye(8, dtype=_f32), wa)
    x8 = edge_attr.reshape(E8, C)
    e2 = _te(x8, m).reshape(E8, 8, 2).reshape(E, 2)

    src = edge_index[0].astype(jnp.int32)
    dst = edge_index[1].astype(jnp.int32)
    src3 = src.reshape(NW, CH, CK)
    dst3 = dst.reshape(NW, CH, CK)
    gidx3 = jnp.concatenate([src3[:, :, :1], src3], axis=2)
    dful3 = jnp.concatenate([jnp.full_like(dst3[:, :, :1], JUNK), dst3],
                            axis=2)
    ea0 = e2[:, 0].reshape(NW, CH, CK)
    ea1 = e2[:, 1].reshape(NW, CH, CK)

    asrc0 = a_src0.reshape(1, C)
    adst0 = a_dst0.reshape(1, C)
    asrc1 = a_src1.reshape(1, C)
    adst1 = a_dst1.reshape(1, C)

    hs0, ss0, sd0 = _t0(xp, emb, rel8, time_w, W0, asrc0, adst0)
    agg0 = _edge(hs0, ss0.reshape(NP), sd0.reshape(NP), gidx3, src3, dst3,
                 dful3, ea0)
    hs1, ss1, sd1 = _t1(agg0[0], agg0[1], hs0, W1, asrc1, adst1)
    agg1 = _edge(hs1, ss1.reshape(NP), sd1.reshape(NP), gidx3, src3, dst3,
                 dful3, ea1)
    return _t2(agg1[0, :B], agg1[1, :B], hs1[:B], head_w,
               head_b.reshape(1, 1))


# speedup vs baseline: 9.0906x; 1.1044x over previous
import functools
import math

import jax
import jax.numpy as jnp
import numpy as np
from jax import lax
from jax.experimental import pallas as pl
from jax.experimental.pallas import tpu as pltpu
from jax.experimental.pallas import tpu_sc as plsc

N, E, C, CE, B, VOC = 10000, 320000, 128, 16, 512, 100000
NP = 10240
NC, NS, L = 2, 16, 16
NW = NC * NS
RT = NP // NW
ET = E // NW
CK = 80
CH = ET // CK
AC = 144
RS = NP // NS

_mesh = plsc.VectorSubcoreMesh(core_axis_name="c", subcore_axis_name="s")
_f32 = jnp.float32
_sc_params = pltpu.CompilerParams(
    use_tc_tiling_on_sc=False, needs_layout_passes=False)


def _prelude_body(nid3, nt3, nb3, st_hbm, emb_hbm, emb_out, rel8_out,
                  idx_v, rows_v, nt_v, nb_v, st_v, rtmp, rel8_v, sem):
    wid = lax.axis_index("s") * NC + lax.axis_index("c")
    base = wid * RT
    pltpu.sync_copy(nid3.at[wid], idx_v)
    pltpu.sync_copy(nt3.at[wid], nt_v)
    pltpu.sync_copy(nb3.at[wid], nb_v)
    pltpu.sync_copy(st_hbm, st_v)
    for j in range(RT // CK):
        pltpu.async_copy(emb_hbm.at[idx_v.at[j]],
                         rows_v.at[pl.ds(j * CK, CK)], sem).wait()
    pltpu.sync_copy(rows_v, emb_out.at[pl.ds(base, RT)])
    inv = _f32(1.0 / 86400.0)
    lane = lax.iota(jnp.int32, L)
    for t in range(RT // L):
        nb = nb_v[pl.ds(t * L, L)]
        s = plsc.load_gather(st_v, [nb])
        r = (s - nt_v[pl.ds(t * L, L)]) * inv
        rtmp[...] = r
        for q in range(8):
            rq = plsc.load_gather(rtmp, [2 * q + lane // 8])
            rel8_v[pl.ds(t * 128 + q * L, L)] = rq
    pltpu.sync_copy(rel8_v, rel8_out.at[pl.ds(base * 8, RT * 8)])


_prelude = functools.partial(
    pl.kernel, _prelude_body, mesh=_mesh, compiler_params=_sc_params,
    out_type=(jax.ShapeDtypeStruct((NP, C), _f32),
              jax.ShapeDtypeStruct((NP * 8,), _f32)),
    scratch_types=[
        pltpu.VMEM((RT // CK, CK), jnp.int32),
        pltpu.VMEM((RT, C), _f32),
        pltpu.VMEM((RT,), _f32),
        pltpu.VMEM((RT,), jnp.int32),
        pltpu.VMEM((B,), _f32),
        pltpu.VMEM((L,), _f32),
        pltpu.VMEM((RT * 8,), _f32),
        pltpu.SemaphoreType.DMA,
    ],
)()


CK1 = CK + 1
JUNK = NP - 1


PK = 352
OG, OD, OS, OT = 0, 96, 192, 272


def _edge_body(hs_hbm, ss_hbm, sd_hbm, pack3, eatt3,
               agg_out, sst, sdt, pk, eb, wbuf, scb,
               agg_sh, sem):
    cid = lax.axis_index("c")
    sid = lax.axis_index("s")
    wid = sid * NC + cid
    zero = jnp.zeros((L,), _f32)
    for r in range(CK1):
        for k in range(AC // L):
            scb[r, pl.ds(k * L, L)] = zero
    for b in range(RS // CK):
        pltpu.sync_copy(scb.at[pl.ds(0, CK)],
                        agg_sh.at[pl.ds(sid * RS + b * CK, CK)])
    plsc.subcore_barrier()
    pltpu.sync_copy(ss_hbm, sst)
    pltpu.sync_copy(sd_hbm, sdt)

    def chunk(j, carry):
        pltpu.sync_copy(pack3.at[wid, j], pk)
        cp = pltpu.async_copy(hs_hbm.at[pk.at[pl.ds(OG, CK1)]], scb, sem)
        pltpu.sync_copy(eatt3.at[wid, j], eb)
        for g in range(CK // L):
            sv = pk[pl.ds(OS + g * L, L)]
            dv = pk[pl.ds(OT + g * L, L)]
            ev = eb[pl.ds(g * L, L)]
            lg = plsc.load_gather(sst, [sv]) + plsc.load_gather(sdt, [dv]) + ev
            lg = jnp.maximum(lg, lg * _f32(0.2))
            wbuf[pl.ds(g * L, L)] = jnp.exp(lg)
        cp.wait()

        def scale(r, c):
            for u in range(4):
                row = 4 * r + u + 1
                wsp = plsc.load_gather(
                    wbuf, [jnp.full((L,), 4 * r + u, jnp.int32)])
                for k in range(AC // L):
                    scb[row, pl.ds(k * L, L)] = (
                        scb[row, pl.ds(k * L, L)] * wsp)
            return c

        lax.fori_loop(0, CK // 4, scale, 0)
        pltpu.sync_copy(scb, agg_sh.at[pk.at[pl.ds(OD, CK1)]], add=True)
        return carry

    lax.fori_loop(0, CH, chunk, 0)
    plsc.subcore_barrier()
    for b in range(RS // CK):
        s0 = sid * RS + b * CK
        pltpu.sync_copy(agg_sh.at[pl.ds(s0, CK)], scb.at[pl.ds(0, CK)])
        pltpu.sync_copy(scb.at[pl.ds(0, CK)], agg_out.at[cid, pl.ds(s0, CK)])


_edge = functools.partial(
    pl.kernel, _edge_body, mesh=_mesh, compiler_params=_sc_params,
    out_type=jax.ShapeDtypeStruct((NC, NP, AC), _f32),
    scratch_types=[
        pltpu.VMEM((NP,), _f32),
        pltpu.VMEM((NP,), _f32),
        pltpu.VMEM((PK,), jnp.int32),
        pltpu.VMEM((CK,), _f32),
        pltpu.VMEM((CK,), _f32),
        pltpu.VMEM((CK1, AC), _f32),
        pltpu.VMEM_SHARED((NP, AC), _f32),
        pltpu.SemaphoreType.DMA,
    ],
)()


RB = 512

def _t0_body(x_r, emb_r, rel8_r, tw_r, w0_r, asrc_r, adst_r,
             hs_r, ss_r, sd_r):
    col = lax.broadcasted_iota(jnp.int32, (1, 8), 1)
    f8 = jnp.exp2((col % 4).astype(_f32))
    ph = jnp.where(col >= 4, _f32(0.5 * math.pi), _f32(0.0))
    tf = jnp.sin(rel8_r[...] * f8 + ph)
    h = x_r[...] + emb_r[...] + jnp.dot(tf, tw_r[...],
                                        preferred_element_type=_f32)
    hs = jnp.dot(h, w0_r[...], preferred_element_type=_f32)
    ac_col = lax.broadcasted_iota(jnp.int32, (1, AC), 1)
    hs_r[...] = (jnp.pad(hs, ((0, 0), (0, AC - C)))
                 + jnp.where(ac_col == C, _f32(1.0), _f32(0.0)))
    ss_r[...] = jnp.sum(hs * asrc_r[...], axis=1, keepdims=True)
    sd_r[...] = jnp.sum(hs * adst_r[...], axis=1, keepdims=True)


def _t0(xp, emb, rel8, tw, w0, asrc, adst):
    g = NP // RB
    return pl.pallas_call(
        _t0_body,
        grid=(g,),
        in_specs=[
            pl.BlockSpec((RB, C), lambda i: (i, 0)),
            pl.BlockSpec((RB, C), lambda i: (i, 0)),
            pl.BlockSpec((RB, 8), lambda i: (i, 0)),
            pl.BlockSpec((8, C), lambda i: (0, 0)),
            pl.BlockSpec((C, C), lambda i: (0, 0)),
            pl.BlockSpec((1, C), lambda i: (0, 0)),
            pl.BlockSpec((1, C), lambda i: (0, 0)),
        ],
        out_specs=[
            pl.BlockSpec((RB, AC), lambda i: (i, 0)),
            pl.BlockSpec((RB, 1), lambda i: (i, 0)),
            pl.BlockSpec((RB, 1), lambda i: (i, 0)),
        ],
        out_shape=[
            jax.ShapeDtypeStruct((NP, AC), _f32),
            jax.ShapeDtypeStruct((NP, 1), _f32),
            jax.ShapeDtypeStruct((NP, 1), _f32),
        ],
    )(xp, emb, rel8, tw, w0, asrc, adst)


def _t1_body(a0_r, a1_r, hs0_r, w1_r, asrc_r, adst_r, hs_r, ss_r, sd_r):
    num = a0_r[:, :C] + a1_r[:, :C]
    den = a0_r[:, C:C + 1] + a1_r[:, C:C + 1]
    h = jnp.maximum(num / (den + _f32(1e-16)) + hs0_r[:, :C], _f32(0.0))
    hs = jnp.dot(h, w1_r[...], preferred_element_type=_f32)
    ac_col = lax.broadcasted_iota(jnp.int32, (1, AC), 1)
    hs_r[...] = (jnp.pad(hs, ((0, 0), (0, AC - C)))
                 + jnp.where(ac_col == C, _f32(1.0), _f32(0.0)))
    ss_r[...] = jnp.sum(hs * asrc_r[...], axis=1, keepdims=True)
    sd_r[...] = jnp.sum(hs * adst_r[...], axis=1, keepdims=True)


def _t1(a0, a1, hs0, w1, asrc, adst):
    g = NP // RB
    return pl.pallas_call(
        _t1_body,
        grid=(g,),
        in_specs=[
            pl.BlockSpec((RB, AC), lambda i: (i, 0)),
            pl.BlockSpec((RB, AC), lambda i: (i, 0)),
            pl.BlockSpec((RB, AC), lambda i: (i, 0)),
            pl.BlockSpec((C, C), lambda i: (0, 0)),
            pl.BlockSpec((1, C), lambda i: (0, 0)),
            pl.BlockSpec((1, C), lambda i: (0, 0)),
        ],
        out_specs=[
            pl.BlockSpec((RB, AC), lambda i: (i, 0)),
            pl.BlockSpec((RB, 1), lambda i: (i, 0)),
            pl.BlockSpec((RB, 1), lambda i: (i, 0)),
        ],
        out_shape=[
            jax.ShapeDtypeStruct((NP, AC), _f32),
            jax.ShapeDtypeStruct((NP, 1), _f32),
            jax.ShapeDtypeStruct((NP, 1), _f32),
        ],
    )(a0, a1, hs0, w1, asrc, adst)


def _t2_body(a0_r, a1_r, hs1_r, hw_r, hb_r, out_r):
    num = a0_r[:, :C] + a1_r[:, :C]
    den = a0_r[:, C:C + 1] + a1_r[:, C:C + 1]
    h = jnp.maximum(num / (den + _f32(1e-16)) + hs1_r[:, :C], _f32(0.0))
    z = jnp.dot(h, hw_r[...], preferred_element_type=_f32) + hb_r[...]
    out_r[...] = jax.nn.sigmoid(z)


def _t2(a0, a1, hs1b, hw, hb):
    return pl.pallas_call(
        _t2_body,
        out_shape=jax.ShapeDtypeStruct((B, 1), _f32),
    )(a0, a1, hs1b, hw, hb)


E8 = E // 8


def _te_body(x8_r, m_r, o_r):
    o_r[...] = jnp.dot(x8_r[...], m_r[...], preferred_element_type=_f32)


def _te(x8, m):
    g = 10
    rb = E8 // g
    return pl.pallas_call(
        _te_body,
        grid=(g,),
        in_specs=[
            pl.BlockSpec((rb, C), lambda i: (i, 0)),
            pl.BlockSpec((C, CE), lambda i: (0, 0)),
        ],
        out_specs=pl.BlockSpec((rb, CE), lambda i: (i, 0)),
        out_shape=jax.ShapeDtypeStruct((E8, CE), _f32),
    )(x8, m)


def kernel(x, edge_index, edge_attr, node_time, node_batch, seed_time, n_id,
           emb_table, time_w, W0, We0, a_src0, a_dst0, a_e0, W1, We1, a_src1,
           a_dst1, a_e1, head_w, head_b):
    pad_n = NP - N
    xp = jnp.pad(x, ((0, pad_n), (0, 0)))
    ntp = jnp.pad(node_time, (0, pad_n))
    nbp = jnp.pad(node_batch, (0, pad_n)).astype(jnp.int32)
    nidp = jnp.pad(n_id, (0, pad_n)).astype(jnp.int32)

    nid3 = nidp.reshape(NW, RT // CK, CK)
    nt3 = ntp.reshape(NW, RT)
    nb3 = nbp.reshape(NW, RT)

    emb, rel8f = _prelude(nid3, nt3, nb3, seed_time, emb_table)
    rel8 = rel8f.reshape(NP, 8)

    wa = jnp.stack([We0 @ a_e0, We1 @ a_e1], axis=1)
    m = jnp.kron(jnp.eye(8, dtype=_f32), wa)
    x8 = edge_attr.reshape(E8, C)
    e2 = _te(x8, m).reshape(E8, 8, 2).reshape(E, 2)

    src = edge_index[0].astype(jnp.int32)
    dst = edge_index[1].astype(jnp.int32)
    src3 = src.reshape(NW, CH, CK)
    dst3 = dst.reshape(NW, CH, CK)
    padz = jnp.zeros((NW, CH, 15), jnp.int32)
    pack3 = jnp.concatenate(
        [src3[:, :, :1], src3, padz,
         jnp.full_like(dst3[:, :, :1], JUNK), dst3, padz,
         src3,
         dst3], axis=2)
    ea0 = e2[:, 0].reshape(NW, CH, CK)
    ea1 = e2[:, 1].reshape(NW, CH, CK)

    asrc0 = a_src0.reshape(1, C)
    adst0 = a_dst0.reshape(1, C)
    asrc1 = a_src1.reshape(1, C)
    adst1 = a_dst1.reshape(1, C)

    hs0, ss0, sd0 = _t0(xp, emb, rel8, time_w, W0, asrc0, adst0)
    agg0 = _edge(hs0, ss0.reshape(NP), sd0.reshape(NP), pack3, ea0)
    hs1, ss1, sd1 = _t1(agg0[0], agg0[1], hs0, W1, asrc1, adst1)
    agg1 = _edge(hs1, ss1.reshape(NP), sd1.reshape(NP), pack3, ea1)
    return _t2(agg1[0, :B], agg1[1, :B], hs1[:B], head_w,
               head_b.reshape(1, 1))
